# baseline (speedup 1.0000x reference)
import functools

import jax
import jax.numpy as jnp
from jax import lax
from jax.experimental import pallas as pl
from jax.experimental.pallas import tpu as pltpu
from jax.experimental.pallas import tpu_sc as plsc

EPS = 1e-5
K = 20
CP = 128
NEG_INF = float("-inf")


def _knn_body(x_ref, wa_ref, idx_ref, ctr_ref, pd_ref, *, R, N):
    b = pl.program_id(0)
    r = pl.program_id(1)
    x_all = x_ref[0]
    rows = x_ref[0, pl.ds(r * R, R), :]

    rows_bf = rows.astype(jnp.bfloat16)
    x_bf = x_all.astype(jnp.bfloat16)
    g = lax.dot_general(rows_bf, x_bf, (((1,), (1,)), ((), ())),
                        preferred_element_type=jnp.float32)
    xx_all = jnp.sum(x_all * x_all, axis=1)
    xx_rows = jnp.sum(rows * rows, axis=1)
    pd_ref[...] = 2.0 * g - xx_rows[:, None] - xx_all[None, :]

    iota = lax.broadcasted_iota(jnp.int32, (R, N), 1)
    kcol = lax.broadcasted_iota(jnp.int32, (R, K), 1)

    def body(t, carry):
        idx_acc, vprev, aprev = carry
        pd = pd_ref[...]
        live = (pd < vprev) | ((pd == vprev) & (iota > aprev))
        pdm = jnp.where(live, pd, NEG_INF)
        m = jnp.max(pdm, axis=1, keepdims=True)
        cand = jnp.where(pdm == m, iota, N)
        am = jnp.min(cand, axis=1, keepdims=True)
        return (jnp.where(kcol == t, am, idx_acc), m, am)

    init = (jnp.zeros((R, K), jnp.int32),
            jnp.full((R, 1), jnp.inf, jnp.float32),
            jnp.full((R, 1), -1, jnp.int32))
    idx_acc, _, _ = lax.fori_loop(0, K, body, init)
    idx_ref[0] = idx_acc + b * N

    ctr_ref[0] = lax.dot_general(rows_bf, wa_ref[...], (((1,), (0,)), ((), ())),
                                 preferred_element_type=jnp.float32)


def _knn_proj(x, wa, *, R=512):
    B, N, _ = x.shape
    O = wa.shape[1]
    return pl.pallas_call(
        functools.partial(_knn_body, R=R, N=N),
        grid=(B, N // R),
        in_specs=[
            pl.BlockSpec((1, N, CP), lambda b, r: (b, 0, 0)),
            pl.BlockSpec((CP, O), lambda b, r: (0, 0)),
        ],
        out_specs=[
            pl.BlockSpec((1, R, K), lambda b, r: (b, r, 0)),
            pl.BlockSpec((1, R, O), lambda b, r: (b, r, 0)),
        ],
        out_shape=[
            jax.ShapeDtypeStruct((B, N, K), jnp.int32),
            jax.ShapeDtypeStruct((B, N, O), jnp.float32),
        ],
        scratch_shapes=[pltpu.VMEM((R, N), jnp.float32)],
    )(x, wa)


def _sc_gather(x2, idx2):
    TOT = x2.shape[0]
    NW = 32
    NPW = TOT // NW
    CH = 64
    NCH = NPW // CH
    NB = 4
    mesh = plsc.VectorSubcoreMesh(core_axis_name="c", subcore_axis_name="s")

    @functools.partial(
        pl.kernel,
        mesh=mesh,
        out_type=jax.ShapeDtypeStruct((TOT, K, CP), jnp.float32),
        scratch_types=[
            pltpu.VMEM((CH, K), jnp.int32),
            pltpu.VMEM((NB, K, CP), jnp.float32),
            pltpu.SemaphoreType.DMA((NB,)),
            pltpu.SemaphoreType.DMA((NB,)),
        ],
    )
    def kern(x_hbm, idx_hbm, out_hbm, idx_v, rows_v, gsem, ssem):
        wid = lax.axis_index("s") * 2 + lax.axis_index("c")

        def chunk_body(ci, _):
            base = wid * NPW + ci * CH
            pltpu.sync_copy(idx_hbm.at[pl.ds(base, CH)], idx_v)

            gathers = {}
            stores = {}
            for j in range(NB - 1):
                gathers[j] = pltpu.async_copy(
                    x_hbm.at[idx_v.at[j]], rows_v.at[j % NB], gsem.at[j % NB])
            for i in range(CH):
                b = i % NB
                gathers.pop(i).wait()
                stores[i] = pltpu.async_copy(
                    rows_v.at[b], out_hbm.at[base + i], ssem.at[b])
                nxt = i + NB - 1
                if nxt < CH:
                    nb_ = nxt % NB
                    if nxt - NB >= 0:
                        stores.pop(nxt - NB).wait()
                    gathers[nxt] = pltpu.async_copy(
                        x_hbm.at[idx_v.at[nxt]], rows_v.at[nb_], gsem.at[nb_])
            for i in sorted(stores):
                stores.pop(i).wait()
            return 0

        lax.fori_loop(0, NCH, chunk_body, 0)

    return kern(x2, idx2)


def _edge_body(g_ref, x_ref, ctr_ref, wb_ref, gam_ref, bet_ref, out_ref,
               *, R, O, OPAD):
    x_rows = x_ref[0]
    ctr = ctr_ref[0]
    gam = gam_ref[...][None, :]
    bet = bet_ref[...][None, :]
    inv = jnp.sqrt(jnp.float32(1.0 + EPS))

    acc = None
    for k in range(K):
        ek = g_ref[0, :, k, :]
        d = (ek - x_rows).astype(jnp.bfloat16)
        zk = lax.dot_general(d, wb_ref[...], (((1,), (0,)), ((), ())),
                             preferred_element_type=jnp.float32)
        t = (ctr + zk) / inv * gam + bet
        t = jnp.where(t >= 0.0, t, 0.2 * t)
        acc = t if acc is None else jnp.maximum(acc, t)

    if OPAD != O:
        acc = jnp.concatenate(
            [acc, jnp.zeros((R, OPAD - O), jnp.float32)], axis=1)
    out_ref[0] = acc


def _edge_tc(gath, x, ctr, wb, gam, bet, OPAD, *, R=256):
    B, N, _ = x.shape
    O = ctr.shape[2]
    vec = lambda n: pl.BlockSpec((n,), lambda b, r: (0,))
    return pl.pallas_call(
        functools.partial(_edge_body, R=R, O=O, OPAD=OPAD),
        grid=(B, N // R),
        in_specs=[
            pl.BlockSpec((1, R, K, CP), lambda b, r: (b, r, 0, 0)),
            pl.BlockSpec((1, R, CP), lambda b, r: (b, r, 0)),
            pl.BlockSpec((1, R, O), lambda b, r: (b, r, 0)),
            pl.BlockSpec((CP, O), lambda b, r: (0, 0)),
            vec(O), vec(O),
        ],
        out_specs=pl.BlockSpec((1, R, OPAD), lambda b, r: (b, r, 0)),
        out_shape=jax.ShapeDtypeStruct((B, N, OPAD), jnp.float32),
    )(gath, x, ctr, wb, gam, bet)


def _head_body(x1_ref, x2_ref, x3_ref, x4_ref, w5_ref, a5_ref, c5_ref,
               fc1_ref, fb1_ref, a6_ref, c6_ref,
               fc2_ref, fb2_ref, a7_ref, c7_ref, out_ref, *, N):
    xc = jnp.concatenate(
        [x1_ref[0][:, :64], x2_ref[0][:, :64],
         x3_ref[0][:, :128], x4_ref[0]], axis=1)
    h5 = lax.dot_general(xc.astype(jnp.bfloat16),
                         w5_ref[...].astype(jnp.bfloat16),
                         (((1,), (0,)), ((), ())),
                         preferred_element_type=jnp.float32)
    inv = jnp.sqrt(jnp.float32(1.0 + EPS))
    h5 = h5 / inv * a5_ref[...][None, :] + c5_ref[...][None, :]
    h5 = jnp.where(h5 >= 0.0, h5, 0.2 * h5)
    mx = jnp.max(h5, axis=0)
    mn = jnp.sum(h5, axis=0) * (1.0 / N)
    feat = jnp.concatenate([mx, mn])[None, :]
    h = lax.dot_general(feat.astype(jnp.bfloat16),
                        fc1_ref[...].astype(jnp.bfloat16),
                        (((1,), (0,)), ((), ())),
                        preferred_element_type=jnp.float32)
    h = (h + fb1_ref[...][None, :]) / inv * a6_ref[...][None, :] + c6_ref[...][None, :]
    h = jnp.where(h >= 0.0, h, 0.2 * h)
    o = lax.dot_general(h.astype(jnp.bfloat16),
                        fc2_ref[...].astype(jnp.bfloat16),
                        (((1,), (0,)), ((), ())),
                        preferred_element_type=jnp.float32)
    out_ref[0] = (o + fb2_ref[...][None, :]) / inv * a7_ref[...][None, :] + c7_ref[...][None, :]


def _head(x1, x2, x3, x4, w5t, a5, c5, fc1t, fc1_b, a6, c6, fc2t, fc2_b, a7, c7):
    B, N, _ = x1.shape
    vec = lambda n: pl.BlockSpec((n,), lambda b: (0,))
    return pl.pallas_call(
        functools.partial(_head_body, N=N),
        grid=(B,),
        in_specs=[
            pl.BlockSpec((1, N, x1.shape[2]), lambda b: (b, 0, 0)),
            pl.BlockSpec((1, N, x2.shape[2]), lambda b: (b, 0, 0)),
            pl.BlockSpec((1, N, x3.shape[2]), lambda b: (b, 0, 0)),
            pl.BlockSpec((1, N, x4.shape[2]), lambda b: (b, 0, 0)),
            pl.BlockSpec((512, 1024), lambda b: (0, 0)),
            vec(1024), vec(1024),
            pl.BlockSpec((2048, 512), lambda b: (0, 0)),
            vec(512), vec(512), vec(512),
            pl.BlockSpec((512, 256), lambda b: (0, 0)),
            vec(256), vec(256), vec(256),
        ],
        out_specs=pl.BlockSpec((1, 1, 256), lambda b: (b, 0, 0)),
        out_shape=jax.ShapeDtypeStruct((B, 1, 256), jnp.float32),
    )(x1, x2, x3, x4, w5t, a5, c5, fc1t, fc1_b, a6, c6, fc2t, fc2_b, a7, c7).reshape(B, 256)


def _edge_layer(x, W, g, b, OPAD):
    B, N, _ = x.shape
    O, C2 = W.shape
    C = C2 // 2
    wa = jnp.pad(jnp.transpose(W[:, :C]), ((0, CP - C), (0, 0))).astype(jnp.bfloat16)
    wb = jnp.pad(jnp.transpose(W[:, C:]), ((0, CP - C), (0, 0))).astype(jnp.bfloat16)
    idx, ctr = _knn_proj(x, wa)
    gath = _sc_gather(x.reshape(B * N, CP), idx.reshape(B * N, K))
    return _edge_tc(gath.reshape(B, N, K, CP), x, ctr, wb, g, b, OPAD)


def kernel(points, W1, g1, b1, W2, g2, b2, W3, g3, b3, W4, g4, b4,
           W5, g5, b5, fc1_W, fc1_b, g6, b6, fc2_W, fc2_b, g7, b7):
    B, N, _ = points.shape
    x0 = jnp.pad(points[:, :, :3].astype(jnp.float32),
                 ((0, 0), (0, 0), (0, CP - 3)))

    x1 = _edge_layer(x0, W1, g1, b1, 128)
    x2 = _edge_layer(x1, W2, g2, b2, 128)
    x3 = _edge_layer(x2, W3, g3, b3, 128)
    x4 = _edge_layer(x3, W4, g4, b4, 256)

    return _head(
        x1, x2, x3, x4,
        jnp.transpose(W5), g5, b5,
        jnp.transpose(fc1_W), fc1_b, g6, b6,
        jnp.transpose(fc2_W), fc2_b, g7, b7,
    )


# speedup vs baseline: 1.3883x; 1.3883x over previous
import functools

import jax
import jax.numpy as jnp
from jax import lax
from jax.experimental import pallas as pl
from jax.experimental.pallas import tpu as pltpu
from jax.experimental.pallas import tpu_sc as plsc

EPS = 1e-5
K = 20
CP = 128
NEG_INF = float("-inf")


def _knn_body(x_ref, wa_ref, idx_ref, ctr_ref, pd_ref, *, R, N):
    b = pl.program_id(0)
    r = pl.program_id(1)
    x_all = x_ref[0]
    rows = x_ref[0, pl.ds(r * R, R), :]

    rows_bf = rows.astype(jnp.bfloat16)
    x_bf = x_all.astype(jnp.bfloat16)
    g = lax.dot_general(rows_bf, x_bf, (((1,), (1,)), ((), ())),
                        preferred_element_type=jnp.float32)
    xx_all = jnp.sum(x_all * x_all, axis=1)
    xx_rows = jnp.sum(rows * rows, axis=1)
    pd_ref[...] = 2.0 * g - xx_rows[:, None] - xx_all[None, :]

    iota = lax.broadcasted_iota(jnp.int32, (R, N), 1)
    kcol = lax.broadcasted_iota(jnp.int32, (R, K), 1)

    def body(t, idx_acc):
        pd = pd_ref[...]
        m = jnp.max(pd, axis=1, keepdims=True)
        cand = jnp.where(pd == m, iota, N)
        am = jnp.min(cand, axis=1, keepdims=True)
        pd_ref[...] = jnp.where(iota == am, NEG_INF, pd)
        return jnp.where(kcol == t, am, idx_acc)

    idx_acc = lax.fori_loop(0, K, body, jnp.zeros((R, K), jnp.int32))
    idx_ref[0] = idx_acc + b * N

    ctr_ref[0] = lax.dot_general(rows_bf, wa_ref[...], (((1,), (0,)), ((), ())),
                                 preferred_element_type=jnp.float32)


def _knn_proj(x, wa, *, R=2048):
    B, N, _ = x.shape
    O = wa.shape[1]
    return pl.pallas_call(
        functools.partial(_knn_body, R=R, N=N),
        grid=(B, N // R),
        in_specs=[
            pl.BlockSpec((1, N, CP), lambda b, r: (b, 0, 0)),
            pl.BlockSpec((CP, O), lambda b, r: (0, 0)),
        ],
        out_specs=[
            pl.BlockSpec((1, R, K), lambda b, r: (b, r, 0)),
            pl.BlockSpec((1, R, O), lambda b, r: (b, r, 0)),
        ],
        out_shape=[
            jax.ShapeDtypeStruct((B, N, K), jnp.int32),
            jax.ShapeDtypeStruct((B, N, O), jnp.float32),
        ],
        scratch_shapes=[pltpu.VMEM((R, N), jnp.float32)],
    )(x, wa)


def _sc_gather(x2, idx2):
    TOT = x2.shape[0]
    NW = 32
    NPW = TOT // NW
    CH = 64
    NCH = NPW // CH
    NB = 4
    mesh = plsc.VectorSubcoreMesh(core_axis_name="c", subcore_axis_name="s")

    @functools.partial(
        pl.kernel,
        mesh=mesh,
        out_type=jax.ShapeDtypeStruct((TOT, K, CP), jnp.float32),
        scratch_types=[
            pltpu.VMEM((CH, K), jnp.int32),
            pltpu.VMEM((NB, K, CP), jnp.float32),
            pltpu.SemaphoreType.DMA((NB,)),
            pltpu.SemaphoreType.DMA((NB,)),
        ],
    )
    def kern(x_hbm, idx_hbm, out_hbm, idx_v, rows_v, gsem, ssem):
        wid = lax.axis_index("s") * 2 + lax.axis_index("c")

        def chunk_body(ci, _):
            base = wid * NPW + ci * CH
            pltpu.sync_copy(idx_hbm.at[pl.ds(base, CH)], idx_v)

            gathers = {}
            stores = {}
            for j in range(NB - 1):
                gathers[j] = pltpu.async_copy(
                    x_hbm.at[idx_v.at[j]], rows_v.at[j % NB], gsem.at[j % NB])
            for i in range(CH):
                b = i % NB
                gathers.pop(i).wait()
                stores[i] = pltpu.async_copy(
                    rows_v.at[b], out_hbm.at[base + i], ssem.at[b])
                nxt = i + NB - 1
                if nxt < CH:
                    nb_ = nxt % NB
                    if nxt - NB >= 0:
                        stores.pop(nxt - NB).wait()
                    gathers[nxt] = pltpu.async_copy(
                        x_hbm.at[idx_v.at[nxt]], rows_v.at[nb_], gsem.at[nb_])
            for i in sorted(stores):
                stores.pop(i).wait()
            return 0

        lax.fori_loop(0, NCH, chunk_body, 0)

    return kern(x2, idx2)


def _edge_body(g_ref, x_ref, ctr_ref, wb_ref, gam_ref, bet_ref, out_ref,
               *, R, O, OPAD):
    x_rows = x_ref[0]
    ctr = ctr_ref[0]
    gam = gam_ref[...][None, :]
    bet = bet_ref[...][None, :]
    inv = jnp.sqrt(jnp.float32(1.0 + EPS))

    acc = None
    for k in range(K):
        ek = g_ref[0, :, k, :]
        d = (ek - x_rows).astype(jnp.bfloat16)
        zk = lax.dot_general(d, wb_ref[...], (((1,), (0,)), ((), ())),
                             preferred_element_type=jnp.float32)
        t = (ctr + zk) / inv * gam + bet
        t = jnp.where(t >= 0.0, t, 0.2 * t)
        acc = t if acc is None else jnp.maximum(acc, t)

    if OPAD != O:
        acc = jnp.concatenate(
            [acc, jnp.zeros((R, OPAD - O), jnp.float32)], axis=1)
    out_ref[0] = acc


def _edge_tc(gath, x, ctr, wb, gam, bet, OPAD, *, R=256):
    B, N, _ = x.shape
    O = ctr.shape[2]
    vec = lambda n: pl.BlockSpec((n,), lambda b, r: (0,))
    return pl.pallas_call(
        functools.partial(_edge_body, R=R, O=O, OPAD=OPAD),
        grid=(B, N // R),
        in_specs=[
            pl.BlockSpec((1, R, K, CP), lambda b, r: (b, r, 0, 0)),
            pl.BlockSpec((1, R, CP), lambda b, r: (b, r, 0)),
            pl.BlockSpec((1, R, O), lambda b, r: (b, r, 0)),
            pl.BlockSpec((CP, O), lambda b, r: (0, 0)),
            vec(O), vec(O),
        ],
        out_specs=pl.BlockSpec((1, R, OPAD), lambda b, r: (b, r, 0)),
        out_shape=jax.ShapeDtypeStruct((B, N, OPAD), jnp.float32),
    )(gath, x, ctr, wb, gam, bet)


def _head_body(x1_ref, x2_ref, x3_ref, x4_ref, w5_ref, a5_ref, c5_ref,
               fc1_ref, fb1_ref, a6_ref, c6_ref,
               fc2_ref, fb2_ref, a7_ref, c7_ref, out_ref, *, N):
    xc = jnp.concatenate(
        [x1_ref[0][:, :64], x2_ref[0][:, :64],
         x3_ref[0][:, :128], x4_ref[0]], axis=1)
    h5 = lax.dot_general(xc.astype(jnp.bfloat16),
                         w5_ref[...].astype(jnp.bfloat16),
                         (((1,), (0,)), ((), ())),
                         preferred_element_type=jnp.float32)
    inv = jnp.sqrt(jnp.float32(1.0 + EPS))
    h5 = h5 / inv * a5_ref[...][None, :] + c5_ref[...][None, :]
    h5 = jnp.where(h5 >= 0.0, h5, 0.2 * h5)
    mx = jnp.max(h5, axis=0)
    mn = jnp.sum(h5, axis=0) * (1.0 / N)
    feat = jnp.concatenate([mx, mn])[None, :]
    h = lax.dot_general(feat.astype(jnp.bfloat16),
                        fc1_ref[...].astype(jnp.bfloat16),
                        (((1,), (0,)), ((), ())),
                        preferred_element_type=jnp.float32)
    h = (h + fb1_ref[...][None, :]) / inv * a6_ref[...][None, :] + c6_ref[...][None, :]
    h = jnp.where(h >= 0.0, h, 0.2 * h)
    o = lax.dot_general(h.astype(jnp.bfloat16),
                        fc2_ref[...].astype(jnp.bfloat16),
                        (((1,), (0,)), ((), ())),
                        preferred_element_type=jnp.float32)
    out_ref[0] = (o + fb2_ref[...][None, :]) / inv * a7_ref[...][None, :] + c7_ref[...][None, :]


def _head(x1, x2, x3, x4, w5t, a5, c5, fc1t, fc1_b, a6, c6, fc2t, fc2_b, a7, c7):
    B, N, _ = x1.shape
    vec = lambda n: pl.BlockSpec((n,), lambda b: (0,))
    return pl.pallas_call(
        functools.partial(_head_body, N=N),
        grid=(B,),
        in_specs=[
            pl.BlockSpec((1, N, x1.shape[2]), lambda b: (b, 0, 0)),
            pl.BlockSpec((1, N, x2.shape[2]), lambda b: (b, 0, 0)),
            pl.BlockSpec((1, N, x3.shape[2]), lambda b: (b, 0, 0)),
            pl.BlockSpec((1, N, x4.shape[2]), lambda b: (b, 0, 0)),
            pl.BlockSpec((512, 1024), lambda b: (0, 0)),
            vec(1024), vec(1024),
            pl.BlockSpec((2048, 512), lambda b: (0, 0)),
            vec(512), vec(512), vec(512),
            pl.BlockSpec((512, 256), lambda b: (0, 0)),
            vec(256), vec(256), vec(256),
        ],
        out_specs=pl.BlockSpec((1, 1, 256), lambda b: (b, 0, 0)),
        out_shape=jax.ShapeDtypeStruct((B, 1, 256), jnp.float32),
    )(x1, x2, x3, x4, w5t, a5, c5, fc1t, fc1_b, a6, c6, fc2t, fc2_b, a7, c7).reshape(B, 256)


def _edge_layer(x, W, g, b, OPAD):
    B, N, _ = x.shape
    O, C2 = W.shape
    C = C2 // 2
    wa = jnp.pad(jnp.transpose(W[:, :C]), ((0, CP - C), (0, 0))).astype(jnp.bfloat16)
    wb = jnp.pad(jnp.transpose(W[:, C:]), ((0, CP - C), (0, 0))).astype(jnp.bfloat16)
    idx, ctr = _knn_proj(x, wa)
    gath = _sc_gather(x.reshape(B * N, CP), idx.reshape(B * N, K))
    return _edge_tc(gath.reshape(B, N, K, CP), x, ctr, wb, g, b, OPAD)


def kernel(points, W1, g1, b1, W2, g2, b2, W3, g3, b3, W4, g4, b4,
           W5, g5, b5, fc1_W, fc1_b, g6, b6, fc2_W, fc2_b, g7, b7):
    B, N, _ = points.shape
    x0 = jnp.pad(points[:, :, :3].astype(jnp.float32),
                 ((0, 0), (0, 0), (0, CP - 3)))

    x1 = _edge_layer(x0, W1, g1, b1, 128)
    x2 = _edge_layer(x1, W2, g2, b2, 128)
    x3 = _edge_layer(x2, W3, g3, b3, 128)
    x4 = _edge_layer(x3, W4, g4, b4, 256)

    return _head(
        x1, x2, x3, x4,
        jnp.transpose(W5), g5, b5,
        jnp.transpose(fc1_W), fc1_b, g6, b6,
        jnp.transpose(fc2_W), fc2_b, g7, b7,
    )


# speedup vs baseline: 1.5317x; 1.1033x over previous
import functools

import jax
import jax.numpy as jnp
from jax import lax
from jax.experimental import pallas as pl
from jax.experimental.pallas import tpu as pltpu
from jax.experimental.pallas import tpu_sc as plsc

EPS = 1e-5
K = 20
CP = 128
NEG_INF = float("-inf")


def _knn_body(x_ref, wa_ref, idx_ref, ctr_ref, pd_ref, *, R, N):
    b = pl.program_id(0)
    r = pl.program_id(1)
    x_all = x_ref[0]
    rows = x_ref[0, pl.ds(r * R, R), :]

    rows_bf = rows.astype(jnp.bfloat16)
    x_bf = x_all.astype(jnp.bfloat16)
    g = lax.dot_general(rows_bf, x_bf, (((1,), (1,)), ((), ())),
                        preferred_element_type=jnp.float32)
    xx_all = jnp.sum(x_all * x_all, axis=1)
    xx_rows = jnp.sum(rows * rows, axis=1)
    pd_ref[...] = 2.0 * g - xx_rows[:, None] - xx_all[None, :]

    iota = lax.broadcasted_iota(jnp.int32, (R, N), 1)
    kcol = lax.broadcasted_iota(jnp.int32, (R, K), 1)

    def body(t, idx_acc):
        pd = pd_ref[...]
        m = jnp.max(pd, axis=1, keepdims=True)
        cand = jnp.where(pd == m, iota, N)
        am = jnp.min(cand, axis=1, keepdims=True)
        pd_ref[...] = jnp.where(iota == am, NEG_INF, pd)
        return jnp.where(kcol == t, am, idx_acc)

    idx_acc = lax.fori_loop(0, K, body, jnp.zeros((R, K), jnp.int32))
    idx_ref[0] = idx_acc + b * N

    ctr_ref[0] = lax.dot_general(rows_bf, wa_ref[...], (((1,), (0,)), ((), ())),
                                 preferred_element_type=jnp.float32)


def _knn_proj(x, wa, *, R=2048):
    B, N, _ = x.shape
    O = wa.shape[1]
    return pl.pallas_call(
        functools.partial(_knn_body, R=R, N=N),
        grid=(B, N // R),
        in_specs=[
            pl.BlockSpec((1, N, CP), lambda b, r: (b, 0, 0)),
            pl.BlockSpec((CP, O), lambda b, r: (0, 0)),
        ],
        out_specs=[
            pl.BlockSpec((1, R, K), lambda b, r: (b, r, 0)),
            pl.BlockSpec((1, R, O), lambda b, r: (b, r, 0)),
        ],
        out_shape=[
            jax.ShapeDtypeStruct((B, N, K), jnp.int32),
            jax.ShapeDtypeStruct((B, N, O), jnp.float32),
        ],
        scratch_shapes=[pltpu.VMEM((R, N), jnp.float32)],
    )(x, wa)


def _sc_gather(x2, idx2):
    TOT = x2.shape[0]
    NW = 32
    NPW = TOT // NW
    CH = 64
    NCH = NPW // CH
    NB = 4
    mesh = plsc.VectorSubcoreMesh(core_axis_name="c", subcore_axis_name="s")

    @functools.partial(
        pl.kernel,
        mesh=mesh,
        out_type=jax.ShapeDtypeStruct((TOT, K, CP), jnp.float32),
        scratch_types=[
            pltpu.VMEM((CH, K), jnp.int32),
            pltpu.VMEM((NB, K, CP), jnp.float32),
            pltpu.SemaphoreType.DMA((NB,)),
            pltpu.SemaphoreType.DMA((NB,)),
        ],
    )
    def kern(x_hbm, idx_hbm, out_hbm, idx_v, rows_v, gsem, ssem):
        wid = lax.axis_index("s") * 2 + lax.axis_index("c")

        def chunk_body(ci, _):
            base = wid * NPW + ci * CH
            pltpu.sync_copy(idx_hbm.at[pl.ds(base, CH)], idx_v)

            gathers = {}
            stores = {}
            for j in range(NB - 1):
                gathers[j] = pltpu.async_copy(
                    x_hbm.at[idx_v.at[j]], rows_v.at[j % NB], gsem.at[j % NB])
            for i in range(CH):
                b = i % NB
                gathers.pop(i).wait()
                stores[i] = pltpu.async_copy(
                    rows_v.at[b], out_hbm.at[base + i], ssem.at[b])
                nxt = i + NB - 1
                if nxt < CH:
                    nb_ = nxt % NB
                    if nxt - NB >= 0:
                        stores.pop(nxt - NB).wait()
                    gathers[nxt] = pltpu.async_copy(
                        x_hbm.at[idx_v.at[nxt]], rows_v.at[nb_], gsem.at[nb_])
            for i in sorted(stores):
                stores.pop(i).wait()
            return 0

        lax.fori_loop(0, NCH, chunk_body, 0)

    return kern(x2, idx2)


def _edge_body(g_ref, x_ref, ctr_ref, wb_ref, gam_ref, bet_ref, out_ref,
               *, R, O, OPAD):
    x_rows = x_ref[0]
    ctr = ctr_ref[0]
    gam = gam_ref[...][None, :]
    bet = bet_ref[...][None, :]
    inv = jnp.sqrt(jnp.float32(1.0 + EPS))

    e3 = g_ref[0]
    d3 = (e3 - x_rows[:, None, :]).astype(jnp.bfloat16)
    z3 = lax.dot_general(d3, wb_ref[...], (((2,), (0,)), ((), ())),
                         preferred_element_type=jnp.float32)
    t = (ctr[:, None, :] + z3) / inv * gam[None] + bet[None]
    t = jnp.where(t >= 0.0, t, 0.2 * t)
    acc = jnp.max(t, axis=1)

    if OPAD != O:
        acc = jnp.concatenate(
            [acc, jnp.zeros((R, OPAD - O), jnp.float32)], axis=1)
    out_ref[0] = acc


def _edge_tc(gath, x, ctr, wb, gam, bet, OPAD, *, R=256):
    B, N, _ = x.shape
    O = ctr.shape[2]
    vec = lambda n: pl.BlockSpec((n,), lambda b, r: (0,))
    return pl.pallas_call(
        functools.partial(_edge_body, R=R, O=O, OPAD=OPAD),
        grid=(B, N // R),
        in_specs=[
            pl.BlockSpec((1, R, K, CP), lambda b, r: (b, r, 0, 0)),
            pl.BlockSpec((1, R, CP), lambda b, r: (b, r, 0)),
            pl.BlockSpec((1, R, O), lambda b, r: (b, r, 0)),
            pl.BlockSpec((CP, O), lambda b, r: (0, 0)),
            vec(O), vec(O),
        ],
        out_specs=pl.BlockSpec((1, R, OPAD), lambda b, r: (b, r, 0)),
        out_shape=jax.ShapeDtypeStruct((B, N, OPAD), jnp.float32),
    )(gath, x, ctr, wb, gam, bet)


def _head_body(x1_ref, x2_ref, x3_ref, x4_ref, w5_ref, a5_ref, c5_ref,
               fc1_ref, fb1_ref, a6_ref, c6_ref,
               fc2_ref, fb2_ref, a7_ref, c7_ref, out_ref, *, N):
    xc = jnp.concatenate(
        [x1_ref[0][:, :64], x2_ref[0][:, :64],
         x3_ref[0][:, :128], x4_ref[0]], axis=1)
    h5 = lax.dot_general(xc.astype(jnp.bfloat16),
                         w5_ref[...].astype(jnp.bfloat16),
                         (((1,), (0,)), ((), ())),
                         preferred_element_type=jnp.float32)
    inv = jnp.sqrt(jnp.float32(1.0 + EPS))
    h5 = h5 / inv * a5_ref[...][None, :] + c5_ref[...][None, :]
    h5 = jnp.where(h5 >= 0.0, h5, 0.2 * h5)
    mx = jnp.max(h5, axis=0)
    mn = jnp.sum(h5, axis=0) * (1.0 / N)
    feat = jnp.concatenate([mx, mn])[None, :]
    h = lax.dot_general(feat.astype(jnp.bfloat16),
                        fc1_ref[...].astype(jnp.bfloat16),
                        (((1,), (0,)), ((), ())),
                        preferred_element_type=jnp.float32)
    h = (h + fb1_ref[...][None, :]) / inv * a6_ref[...][None, :] + c6_ref[...][None, :]
    h = jnp.where(h >= 0.0, h, 0.2 * h)
    o = lax.dot_general(h.astype(jnp.bfloat16),
                        fc2_ref[...].astype(jnp.bfloat16),
                        (((1,), (0,)), ((), ())),
                        preferred_element_type=jnp.float32)
    out_ref[0] = (o + fb2_ref[...][None, :]) / inv * a7_ref[...][None, :] + c7_ref[...][None, :]


def _head(x1, x2, x3, x4, w5t, a5, c5, fc1t, fc1_b, a6, c6, fc2t, fc2_b, a7, c7):
    B, N, _ = x1.shape
    vec = lambda n: pl.BlockSpec((n,), lambda b: (0,))
    return pl.pallas_call(
        functools.partial(_head_body, N=N),
        grid=(B,),
        in_specs=[
            pl.BlockSpec((1, N, x1.shape[2]), lambda b: (b, 0, 0)),
            pl.BlockSpec((1, N, x2.shape[2]), lambda b: (b, 0, 0)),
            pl.BlockSpec((1, N, x3.shape[2]), lambda b: (b, 0, 0)),
            pl.BlockSpec((1, N, x4.shape[2]), lambda b: (b, 0, 0)),
            pl.BlockSpec((512, 1024), lambda b: (0, 0)),
            vec(1024), vec(1024),
            pl.BlockSpec((2048, 512), lambda b: (0, 0)),
            vec(512), vec(512), vec(512),
            pl.BlockSpec((512, 256), lambda b: (0, 0)),
            vec(256), vec(256), vec(256),
        ],
        out_specs=pl.BlockSpec((1, 1, 256), lambda b: (b, 0, 0)),
        out_shape=jax.ShapeDtypeStruct((B, 1, 256), jnp.float32),
    )(x1, x2, x3, x4, w5t, a5, c5, fc1t, fc1_b, a6, c6, fc2t, fc2_b, a7, c7).reshape(B, 256)


def _edge_layer(x, W, g, b, OPAD):
    B, N, _ = x.shape
    O, C2 = W.shape
    C = C2 // 2
    wa = jnp.pad(jnp.transpose(W[:, :C]), ((0, CP - C), (0, 0))).astype(jnp.bfloat16)
    wb = jnp.pad(jnp.transpose(W[:, C:]), ((0, CP - C), (0, 0))).astype(jnp.bfloat16)
    idx, ctr = _knn_proj(x, wa)
    gath = _sc_gather(x.reshape(B * N, CP), idx.reshape(B * N, K))
    return _edge_tc(gath.reshape(B, N, K, CP), x, ctr, wb, g, b, OPAD)


def kernel(points, W1, g1, b1, W2, g2, b2, W3, g3, b3, W4, g4, b4,
           W5, g5, b5, fc1_W, fc1_b, g6, b6, fc2_W, fc2_b, g7, b7):
    B, N, _ = points.shape
    x0 = jnp.pad(points[:, :, :3].astype(jnp.float32),
                 ((0, 0), (0, 0), (0, CP - 3)))

    x1 = _edge_layer(x0, W1, g1, b1, 128)
    x2 = _edge_layer(x1, W2, g2, b2, 128)
    x3 = _edge_layer(x2, W3, g3, b3, 128)
    x4 = _edge_layer(x3, W4, g4, b4, 256)

    return _head(
        x1, x2, x3, x4,
        jnp.transpose(W5), g5, b5,
        jnp.transpose(fc1_W), fc1_b, g6, b6,
        jnp.transpose(fc2_W), fc2_b, g7, b7,
    )


# speedup vs baseline: 1.6135x; 1.0534x over previous
import functools

import jax
import jax.numpy as jnp
from jax import lax
from jax.experimental import pallas as pl
from jax.experimental.pallas import tpu as pltpu
from jax.experimental.pallas import tpu_sc as plsc

EPS = 1e-5
K = 20
CP = 128
NEG_INF = float("-inf")


def _knn_body(x_ref, wa_ref, idx_ref, ctr_ref, pd_ref, *, R, N):
    b = pl.program_id(0)
    r = pl.program_id(1)
    x_all = x_ref[0]
    rows = x_ref[0, pl.ds(r * R, R), :]

    rows_bf = rows.astype(jnp.bfloat16)
    x_bf = x_all.astype(jnp.bfloat16)
    g = lax.dot_general(rows_bf, x_bf, (((1,), (1,)), ((), ())),
                        preferred_element_type=jnp.float32)
    xx_all = jnp.sum(x_all * x_all, axis=1)
    xx_rows = jnp.sum(rows * rows, axis=1)
    pd_ref[...] = 2.0 * g - xx_rows[:, None] - xx_all[None, :]

    iota = lax.broadcasted_iota(jnp.int32, (R, N), 1)
    kcol = lax.broadcasted_iota(jnp.int32, (R, K), 1)

    def body(t, idx_acc):
        pd = pd_ref[...]
        m = jnp.max(pd, axis=1, keepdims=True)
        cand = jnp.where(pd == m, iota, N)
        am = jnp.min(cand, axis=1, keepdims=True)
        pd_ref[...] = jnp.where(iota == am, NEG_INF, pd)
        return jnp.where(kcol == t, am, idx_acc)

    idx_acc = lax.fori_loop(0, K, body, jnp.zeros((R, K), jnp.int32))
    idx_ref[0] = idx_acc + b * N

    ctr_ref[0] = lax.dot_general(rows_bf, wa_ref[...], (((1,), (0,)), ((), ())),
                                 preferred_element_type=jnp.float32)


def _knn_proj(x, wa, *, R=2048):
    B, N, _ = x.shape
    O = wa.shape[1]
    return pl.pallas_call(
        functools.partial(_knn_body, R=R, N=N),
        grid=(B, N // R),
        in_specs=[
            pl.BlockSpec((1, N, CP), lambda b, r: (b, 0, 0)),
            pl.BlockSpec((CP, O), lambda b, r: (0, 0)),
        ],
        out_specs=[
            pl.BlockSpec((1, R, K), lambda b, r: (b, r, 0)),
            pl.BlockSpec((1, R, O), lambda b, r: (b, r, 0)),
        ],
        out_shape=[
            jax.ShapeDtypeStruct((B, N, K), jnp.int32),
            jax.ShapeDtypeStruct((B, N, O), jnp.float32),
        ],
        scratch_shapes=[pltpu.VMEM((R, N), jnp.float32)],
    )(x, wa)


def _sc_gather(x2, idx2):
    TOT = x2.shape[0]
    NW = 32
    NPW = TOT // NW
    CH = 128
    NCH = NPW // CH
    NB = 8
    mesh = plsc.VectorSubcoreMesh(core_axis_name="c", subcore_axis_name="s")

    @functools.partial(
        pl.kernel,
        mesh=mesh,
        out_type=jax.ShapeDtypeStruct((TOT, K, CP), jnp.float32),
        scratch_types=[
            pltpu.VMEM((CH, K), jnp.int32),
            pltpu.VMEM((NB, K, CP), jnp.float32),
            pltpu.SemaphoreType.DMA((NB,)),
            pltpu.SemaphoreType.DMA((NB,)),
        ],
    )
    def kern(x_hbm, idx_hbm, out_hbm, idx_v, rows_v, gsem, ssem):
        wid = lax.axis_index("s") * 2 + lax.axis_index("c")

        def chunk_body(ci, _):
            base = wid * NPW + ci * CH
            pltpu.sync_copy(idx_hbm.at[pl.ds(base, CH)], idx_v)

            gathers = {}
            stores = {}
            for j in range(NB - 1):
                gathers[j] = pltpu.async_copy(
                    x_hbm.at[idx_v.at[j]], rows_v.at[j % NB], gsem.at[j % NB])
            for i in range(CH):
                b = i % NB
                gathers.pop(i).wait()
                stores[i] = pltpu.async_copy(
                    rows_v.at[b], out_hbm.at[base + i], ssem.at[b])
                nxt = i + NB - 1
                if nxt < CH:
                    nb_ = nxt % NB
                    if nxt - NB >= 0:
                        stores.pop(nxt - NB).wait()
                    gathers[nxt] = pltpu.async_copy(
                        x_hbm.at[idx_v.at[nxt]], rows_v.at[nb_], gsem.at[nb_])
            for i in sorted(stores):
                stores.pop(i).wait()
            return 0

        lax.fori_loop(0, NCH, chunk_body, 0)

    return kern(x2, idx2)


def _edge_body(g_ref, x_ref, ctr_ref, wb_ref, gam_ref, bet_ref, out_ref,
               *, R, O, OPAD):
    x_rows = x_ref[0]
    ctr = ctr_ref[0]
    gam = gam_ref[...][None, :]
    bet = bet_ref[...][None, :]
    inv = jnp.sqrt(jnp.float32(1.0 + EPS))

    e3 = g_ref[0]
    d3 = (e3 - x_rows[:, None, :]).astype(jnp.bfloat16)
    z3 = lax.dot_general(d3, wb_ref[...], (((2,), (0,)), ((), ())),
                         preferred_element_type=jnp.float32)
    t = (ctr[:, None, :] + z3) / inv * gam[None] + bet[None]
    t = jnp.where(t >= 0.0, t, 0.2 * t)
    acc = jnp.max(t, axis=1)

    if OPAD != O:
        acc = jnp.concatenate(
            [acc, jnp.zeros((R, OPAD - O), jnp.float32)], axis=1)
    out_ref[0] = acc


def _edge_tc(gath, x, ctr, wb, gam, bet, OPAD, *, R=256):
    B, N, _ = x.shape
    O = ctr.shape[2]
    vec = lambda n: pl.BlockSpec((n,), lambda b, r: (0,))
    return pl.pallas_call(
        functools.partial(_edge_body, R=R, O=O, OPAD=OPAD),
        grid=(B, N // R),
        in_specs=[
            pl.BlockSpec((1, R, K, CP), lambda b, r: (b, r, 0, 0)),
            pl.BlockSpec((1, R, CP), lambda b, r: (b, r, 0)),
            pl.BlockSpec((1, R, O), lambda b, r: (b, r, 0)),
            pl.BlockSpec((CP, O), lambda b, r: (0, 0)),
            vec(O), vec(O),
        ],
        out_specs=pl.BlockSpec((1, R, OPAD), lambda b, r: (b, r, 0)),
        out_shape=jax.ShapeDtypeStruct((B, N, OPAD), jnp.float32),
    )(gath, x, ctr, wb, gam, bet)


def _head_body(x1_ref, x2_ref, x3_ref, x4_ref, w5_ref, a5_ref, c5_ref,
               fc1_ref, fb1_ref, a6_ref, c6_ref,
               fc2_ref, fb2_ref, a7_ref, c7_ref, out_ref, *, N):
    xc = jnp.concatenate(
        [x1_ref[0][:, :64], x2_ref[0][:, :64],
         x3_ref[0][:, :128], x4_ref[0]], axis=1)
    h5 = lax.dot_general(xc.astype(jnp.bfloat16),
                         w5_ref[...].astype(jnp.bfloat16),
                         (((1,), (0,)), ((), ())),
                         preferred_element_type=jnp.float32)
    inv = jnp.sqrt(jnp.float32(1.0 + EPS))
    h5 = h5 / inv * a5_ref[...][None, :] + c5_ref[...][None, :]
    h5 = jnp.where(h5 >= 0.0, h5, 0.2 * h5)
    mx = jnp.max(h5, axis=0)
    mn = jnp.sum(h5, axis=0) * (1.0 / N)
    feat = jnp.concatenate([mx, mn])[None, :]
    h = lax.dot_general(feat.astype(jnp.bfloat16),
                        fc1_ref[...].astype(jnp.bfloat16),
                        (((1,), (0,)), ((), ())),
                        preferred_element_type=jnp.float32)
    h = (h + fb1_ref[...][None, :]) / inv * a6_ref[...][None, :] + c6_ref[...][None, :]
    h = jnp.where(h >= 0.0, h, 0.2 * h)
    o = lax.dot_general(h.astype(jnp.bfloat16),
                        fc2_ref[...].astype(jnp.bfloat16),
                        (((1,), (0,)), ((), ())),
                        preferred_element_type=jnp.float32)
    out_ref[0] = (o + fb2_ref[...][None, :]) / inv * a7_ref[...][None, :] + c7_ref[...][None, :]


def _head(x1, x2, x3, x4, w5t, a5, c5, fc1t, fc1_b, a6, c6, fc2t, fc2_b, a7, c7):
    B, N, _ = x1.shape
    vec = lambda n: pl.BlockSpec((n,), lambda b: (0,))
    return pl.pallas_call(
        functools.partial(_head_body, N=N),
        grid=(B,),
        in_specs=[
            pl.BlockSpec((1, N, x1.shape[2]), lambda b: (b, 0, 0)),
            pl.BlockSpec((1, N, x2.shape[2]), lambda b: (b, 0, 0)),
            pl.BlockSpec((1, N, x3.shape[2]), lambda b: (b, 0, 0)),
            pl.BlockSpec((1, N, x4.shape[2]), lambda b: (b, 0, 0)),
            pl.BlockSpec((512, 1024), lambda b: (0, 0)),
            vec(1024), vec(1024),
            pl.BlockSpec((2048, 512), lambda b: (0, 0)),
            vec(512), vec(512), vec(512),
            pl.BlockSpec((512, 256), lambda b: (0, 0)),
            vec(256), vec(256), vec(256),
        ],
        out_specs=pl.BlockSpec((1, 1, 256), lambda b: (b, 0, 0)),
        out_shape=jax.ShapeDtypeStruct((B, 1, 256), jnp.float32),
    )(x1, x2, x3, x4, w5t, a5, c5, fc1t, fc1_b, a6, c6, fc2t, fc2_b, a7, c7).reshape(B, 256)


def _edge_layer(x, W, g, b, OPAD):
    B, N, _ = x.shape
    O, C2 = W.shape
    C = C2 // 2
    wa = jnp.pad(jnp.transpose(W[:, :C]), ((0, CP - C), (0, 0))).astype(jnp.bfloat16)
    wb = jnp.pad(jnp.transpose(W[:, C:]), ((0, CP - C), (0, 0))).astype(jnp.bfloat16)
    idx, ctr = _knn_proj(x, wa)
    gath = _sc_gather(x.reshape(B * N, CP), idx.reshape(B * N, K))
    return _edge_tc(gath.reshape(B, N, K, CP), x, ctr, wb, g, b, OPAD)


def kernel(points, W1, g1, b1, W2, g2, b2, W3, g3, b3, W4, g4, b4,
           W5, g5, b5, fc1_W, fc1_b, g6, b6, fc2_W, fc2_b, g7, b7):
    B, N, _ = points.shape
    x0 = jnp.pad(points[:, :, :3].astype(jnp.float32),
                 ((0, 0), (0, 0), (0, CP - 3)))

    x1 = _edge_layer(x0, W1, g1, b1, 128)
    x2 = _edge_layer(x1, W2, g2, b2, 128)
    x3 = _edge_layer(x2, W3, g3, b3, 128)
    x4 = _edge_layer(x3, W4, g4, b4, 256)

    return _head(
        x1, x2, x3, x4,
        jnp.transpose(W5), g5, b5,
        jnp.transpose(fc1_W), fc1_b, g6, b6,
        jnp.transpose(fc2_W), fc2_b, g7, b7,
    )


# speedup vs baseline: 1.6359x; 1.0139x over previous
import functools

import jax
import jax.numpy as jnp
from jax import lax
from jax.experimental import pallas as pl
from jax.experimental.pallas import tpu as pltpu
from jax.experimental.pallas import tpu_sc as plsc

EPS = 1e-5
K = 20
CP = 128
NEG_INF = float("-inf")


def _knn_body(x_ref, wa_ref, idx_ref, ctr_ref, pd_ref, *, R, N):
    b = pl.program_id(0)
    r = pl.program_id(1)
    x_all = x_ref[0]
    rows = x_ref[0, pl.ds(r * R, R), :]

    rows_bf = rows.astype(jnp.bfloat16)
    x_bf = x_all.astype(jnp.bfloat16)
    g = lax.dot_general(rows_bf, x_bf, (((1,), (1,)), ((), ())),
                        preferred_element_type=jnp.float32)
    xx_all = jnp.sum(x_all * x_all, axis=1)
    xx_rows = jnp.sum(rows * rows, axis=1)
    pd_ref[...] = 2.0 * g - xx_rows[:, None] - xx_all[None, :]

    iota = lax.broadcasted_iota(jnp.int32, (R, N), 1)
    kcol = lax.broadcasted_iota(jnp.int32, (R, K), 1)

    def body(t, idx_acc):
        pd = pd_ref[...]
        m = jnp.max(pd, axis=1, keepdims=True)
        cand = jnp.where(pd == m, iota, N)
        am = jnp.min(cand, axis=1, keepdims=True)
        pd_ref[...] = jnp.where(iota == am, NEG_INF, pd)
        return jnp.where(kcol == t, am, idx_acc)

    idx_acc = lax.fori_loop(0, K, body, jnp.zeros((R, K), jnp.int32))
    idx_ref[0] = idx_acc + b * N

    ctr_ref[0] = lax.dot_general(rows_bf, wa_ref[...], (((1,), (0,)), ((), ())),
                                 preferred_element_type=jnp.float32)


def _knn_proj(x, wa, *, R=2048):
    B, N, _ = x.shape
    O = wa.shape[1]
    return pl.pallas_call(
        functools.partial(_knn_body, R=R, N=N),
        grid=(B, N // R),
        in_specs=[
            pl.BlockSpec((1, N, CP), lambda b, r: (b, 0, 0)),
            pl.BlockSpec((CP, O), lambda b, r: (0, 0)),
        ],
        out_specs=[
            pl.BlockSpec((1, R, K), lambda b, r: (b, r, 0)),
            pl.BlockSpec((1, R, O), lambda b, r: (b, r, 0)),
        ],
        out_shape=[
            jax.ShapeDtypeStruct((B, N, K), jnp.int32),
            jax.ShapeDtypeStruct((B, N, O), jnp.float32),
        ],
        scratch_shapes=[pltpu.VMEM((R, N), jnp.float32)],
    )(x, wa)


def _sc_gather(x2, idx2):
    TOT = x2.shape[0]
    NW = 32
    NPW = TOT // NW
    CH = min(128, NPW)
    NCH = NPW // CH
    NB = 8
    mesh = plsc.VectorSubcoreMesh(core_axis_name="c", subcore_axis_name="s")

    @functools.partial(
        pl.kernel,
        mesh=mesh,
        out_type=jax.ShapeDtypeStruct((TOT, K, CP), jnp.float32),
        scratch_types=[
            pltpu.VMEM((CH, K), jnp.int32),
            pltpu.VMEM((NB, K, CP), jnp.float32),
            pltpu.SemaphoreType.DMA((NB,)),
            pltpu.SemaphoreType.DMA((NB,)),
        ],
    )
    def kern(x_hbm, idx_hbm, out_hbm, idx_v, rows_v, gsem, ssem):
        wid = lax.axis_index("s") * 2 + lax.axis_index("c")

        def chunk_body(ci, _):
            base = wid * NPW + ci * CH
            pltpu.sync_copy(idx_hbm.at[pl.ds(base, CH)], idx_v)

            gathers = {}
            stores = {}
            for j in range(NB - 1):
                gathers[j] = pltpu.async_copy(
                    x_hbm.at[idx_v.at[j]], rows_v.at[j % NB], gsem.at[j % NB])
            for i in range(CH):
                b = i % NB
                gathers.pop(i).wait()
                stores[i] = pltpu.async_copy(
                    rows_v.at[b], out_hbm.at[base + i], ssem.at[b])
                nxt = i + NB - 1
                if nxt < CH:
                    nb_ = nxt % NB
                    if nxt - NB >= 0:
                        stores.pop(nxt - NB).wait()
                    gathers[nxt] = pltpu.async_copy(
                        x_hbm.at[idx_v.at[nxt]], rows_v.at[nb_], gsem.at[nb_])
            for i in sorted(stores):
                stores.pop(i).wait()
            return 0

        lax.fori_loop(0, NCH, chunk_body, 0)

    return kern(x2, idx2)


def _edge_body(g_ref, x_ref, ctr_ref, wb_ref, gam_ref, bet_ref, out_ref,
               *, R, O, OPAD):
    x_rows = x_ref[0]
    ctr = ctr_ref[0]
    gam = gam_ref[...][None, :]
    bet = bet_ref[...][None, :]
    inv = jnp.sqrt(jnp.float32(1.0 + EPS))

    e3 = g_ref[0]
    d3 = (e3 - x_rows[:, None, :]).astype(jnp.bfloat16)
    z3 = lax.dot_general(d3, wb_ref[...], (((2,), (0,)), ((), ())),
                         preferred_element_type=jnp.float32)
    t = (ctr[:, None, :] + z3) / inv * gam[None] + bet[None]
    t = jnp.where(t >= 0.0, t, 0.2 * t)
    acc = jnp.max(t, axis=1)

    if OPAD != O:
        acc = jnp.concatenate(
            [acc, jnp.zeros((R, OPAD - O), jnp.float32)], axis=1)
    out_ref[0] = acc


def _edge_tc(gath, x, ctr, wb, gam, bet, OPAD, *, R=256):
    B, N, _ = x.shape
    O = ctr.shape[2]
    vec = lambda n: pl.BlockSpec((n,), lambda b, r: (0,))
    return pl.pallas_call(
        functools.partial(_edge_body, R=R, O=O, OPAD=OPAD),
        grid=(B, N // R),
        in_specs=[
            pl.BlockSpec((1, R, K, CP), lambda b, r: (b, r, 0, 0)),
            pl.BlockSpec((1, R, CP), lambda b, r: (b, r, 0)),
            pl.BlockSpec((1, R, O), lambda b, r: (b, r, 0)),
            pl.BlockSpec((CP, O), lambda b, r: (0, 0)),
            vec(O), vec(O),
        ],
        out_specs=pl.BlockSpec((1, R, OPAD), lambda b, r: (b, r, 0)),
        out_shape=jax.ShapeDtypeStruct((B, N, OPAD), jnp.float32),
    )(gath, x, ctr, wb, gam, bet)


def _head_body(x1_ref, x2_ref, x3_ref, x4_ref, w5_ref, a5_ref, c5_ref,
               fc1_ref, fb1_ref, a6_ref, c6_ref,
               fc2_ref, fb2_ref, a7_ref, c7_ref, out_ref, *, N):
    xc = jnp.concatenate(
        [x1_ref[0][:, :64], x2_ref[0][:, :64],
         x3_ref[0][:, :128], x4_ref[0]], axis=1)
    h5 = lax.dot_general(xc.astype(jnp.bfloat16),
                         w5_ref[...].astype(jnp.bfloat16),
                         (((1,), (0,)), ((), ())),
                         preferred_element_type=jnp.float32)
    inv = jnp.sqrt(jnp.float32(1.0 + EPS))
    h5 = h5 / inv * a5_ref[...][None, :] + c5_ref[...][None, :]
    h5 = jnp.where(h5 >= 0.0, h5, 0.2 * h5)
    mx = jnp.max(h5, axis=0)
    mn = jnp.sum(h5, axis=0) * (1.0 / N)
    feat = jnp.concatenate([mx, mn])[None, :]
    h = lax.dot_general(feat.astype(jnp.bfloat16),
                        fc1_ref[...].astype(jnp.bfloat16),
                        (((1,), (0,)), ((), ())),
                        preferred_element_type=jnp.float32)
    h = (h + fb1_ref[...][None, :]) / inv * a6_ref[...][None, :] + c6_ref[...][None, :]
    h = jnp.where(h >= 0.0, h, 0.2 * h)
    o = lax.dot_general(h.astype(jnp.bfloat16),
                        fc2_ref[...].astype(jnp.bfloat16),
                        (((1,), (0,)), ((), ())),
                        preferred_element_type=jnp.float32)
    out_ref[0] = (o + fb2_ref[...][None, :]) / inv * a7_ref[...][None, :] + c7_ref[...][None, :]


def _head(x1, x2, x3, x4, w5t, a5, c5, fc1t, fc1_b, a6, c6, fc2t, fc2_b, a7, c7):
    B, N, _ = x1.shape
    vec = lambda n: pl.BlockSpec((n,), lambda b: (0,))
    return pl.pallas_call(
        functools.partial(_head_body, N=N),
        grid=(B,),
        in_specs=[
            pl.BlockSpec((1, N, x1.shape[2]), lambda b: (b, 0, 0)),
            pl.BlockSpec((1, N, x2.shape[2]), lambda b: (b, 0, 0)),
            pl.BlockSpec((1, N, x3.shape[2]), lambda b: (b, 0, 0)),
            pl.BlockSpec((1, N, x4.shape[2]), lambda b: (b, 0, 0)),
            pl.BlockSpec((512, 1024), lambda b: (0, 0)),
            vec(1024), vec(1024),
            pl.BlockSpec((2048, 512), lambda b: (0, 0)),
            vec(512), vec(512), vec(512),
            pl.BlockSpec((512, 256), lambda b: (0, 0)),
            vec(256), vec(256), vec(256),
        ],
        out_specs=pl.BlockSpec((1, 1, 256), lambda b: (b, 0, 0)),
        out_shape=jax.ShapeDtypeStruct((B, 1, 256), jnp.float32),
    )(x1, x2, x3, x4, w5t, a5, c5, fc1t, fc1_b, a6, c6, fc2t, fc2_b, a7, c7).reshape(B, 256)


def _edge_layer(x, W, g, b, OPAD):
    B, N, _ = x.shape
    O, C2 = W.shape
    C = C2 // 2
    wa = jnp.pad(jnp.transpose(W[:, :C]), ((0, CP - C), (0, 0))).astype(jnp.bfloat16)
    wb = jnp.pad(jnp.transpose(W[:, C:]), ((0, CP - C), (0, 0))).astype(jnp.bfloat16)
    idx, ctr = _knn_proj(x, wa)
    gath = _sc_gather(x.reshape(B * N, CP), idx.reshape(B * N, K))
    return _edge_tc(gath.reshape(B, N, K, CP), x, ctr, wb, g, b, OPAD)


def kernel(points, W1, g1, b1, W2, g2, b2, W3, g3, b3, W4, g4, b4,
           W5, g5, b5, fc1_W, fc1_b, g6, b6, fc2_W, fc2_b, g7, b7):
    B, N, _ = points.shape
    x0 = jnp.pad(points[:, :, :3].astype(jnp.float32),
                 ((0, 0), (0, 0), (0, CP - 3)))

    x1l, x2l, x3l, x4l = [], [], [], []
    for bi in range(B):
        xb = x0[bi:bi + 1]
        xb1 = _edge_layer(xb, W1, g1, b1, 128)
        xb2 = _edge_layer(xb1, W2, g2, b2, 128)
        xb3 = _edge_layer(xb2, W3, g3, b3, 128)
        xb4 = _edge_layer(xb3, W4, g4, b4, 256)
        x1l.append(xb1); x2l.append(xb2); x3l.append(xb3); x4l.append(xb4)
    x1 = jnp.concatenate(x1l, axis=0)
    x2 = jnp.concatenate(x2l, axis=0)
    x3 = jnp.concatenate(x3l, axis=0)
    x4 = jnp.concatenate(x4l, axis=0)

    return _head(
        x1, x2, x3, x4,
        jnp.transpose(W5), g5, b5,
        jnp.transpose(fc1_W), fc1_b, g6, b6,
        jnp.transpose(fc2_W), fc2_b, g7, b7,
    )


# speedup vs baseline: 1.6475x; 1.0070x over previous
import functools

import jax
import jax.numpy as jnp
from jax import lax
from jax.experimental import pallas as pl
from jax.experimental.pallas import tpu as pltpu
from jax.experimental.pallas import tpu_sc as plsc

EPS = 1e-5
K = 20
CP = 128
NEG_INF = float("-inf")


def _knn_body(x_ref, wa_ref, idx_ref, ctr_ref, pd_ref, *, R, N):
    b = pl.program_id(0)
    r = pl.program_id(1)
    x_all = x_ref[0]
    rows = x_ref[0, pl.ds(r * R, R), :]

    rows_bf = rows.astype(jnp.bfloat16)
    x_bf = x_all.astype(jnp.bfloat16)
    g = lax.dot_general(rows_bf, x_bf, (((1,), (1,)), ((), ())),
                        preferred_element_type=jnp.float32)
    xx_all = jnp.sum(x_all * x_all, axis=1)
    xx_rows = jnp.sum(rows * rows, axis=1)
    pd_ref[...] = 2.0 * g - xx_rows[:, None] - xx_all[None, :]

    iota = lax.broadcasted_iota(jnp.int32, (R, N), 1)
    kcol = lax.broadcasted_iota(jnp.int32, (R, K), 1)

    def body(t, idx_acc):
        pd = pd_ref[...]
        m = jnp.max(pd, axis=1, keepdims=True)
        cand = jnp.where(pd == m, iota, N)
        am = jnp.min(cand, axis=1, keepdims=True)
        pd_ref[...] = jnp.where(iota == am, NEG_INF, pd)
        return jnp.where(kcol == t, am, idx_acc)

    idx_acc = lax.fori_loop(0, K, body, jnp.zeros((R, K), jnp.int32))
    idx_ref[0] = idx_acc + b * N

    ctr_ref[0] = lax.dot_general(rows_bf, wa_ref[...], (((1,), (0,)), ((), ())),
                                 preferred_element_type=jnp.float32)


def _knn_proj(x, wa, *, R=2048):
    B, N, _ = x.shape
    O = wa.shape[1]
    return pl.pallas_call(
        functools.partial(_knn_body, R=R, N=N),
        grid=(B, N // R),
        in_specs=[
            pl.BlockSpec((1, N, CP), lambda b, r: (b, 0, 0)),
            pl.BlockSpec((CP, O), lambda b, r: (0, 0)),
        ],
        out_specs=[
            pl.BlockSpec((1, R, K), lambda b, r: (b, r, 0)),
            pl.BlockSpec((1, R, O), lambda b, r: (b, r, 0)),
        ],
        out_shape=[
            jax.ShapeDtypeStruct((B, N, K), jnp.int32),
            jax.ShapeDtypeStruct((B, N, O), jnp.float32),
        ],
        scratch_shapes=[pltpu.VMEM((R, N), jnp.float32)],
    )(x, wa)


def _sc_gather(x2, idx2):
    TOT = x2.shape[0]
    NW = 32
    NPW = TOT // NW
    CH = min(128, NPW)
    NCH = NPW // CH
    NB = 8
    mesh = plsc.VectorSubcoreMesh(core_axis_name="c", subcore_axis_name="s")

    @functools.partial(
        pl.kernel,
        mesh=mesh,
        out_type=jax.ShapeDtypeStruct((TOT, K, CP), jnp.float32),
        scratch_types=[
            pltpu.VMEM((CH, K), jnp.int32),
            pltpu.VMEM((NB, K, CP), jnp.float32),
            pltpu.SemaphoreType.DMA((NB,)),
            pltpu.SemaphoreType.DMA((NB,)),
        ],
    )
    def kern(x_hbm, idx_hbm, out_hbm, idx_v, rows_v, gsem, ssem):
        wid = lax.axis_index("s") * 2 + lax.axis_index("c")

        def chunk_body(ci, _):
            base = wid * NPW + ci * CH
            pltpu.sync_copy(idx_hbm.at[pl.ds(base, CH)], idx_v)

            gathers = {}
            stores = {}
            for j in range(NB - 1):
                gathers[j] = pltpu.async_copy(
                    x_hbm.at[idx_v.at[j]], rows_v.at[j % NB], gsem.at[j % NB])
            for i in range(CH):
                b = i % NB
                gathers.pop(i).wait()
                stores[i] = pltpu.async_copy(
                    rows_v.at[b], out_hbm.at[base + i], ssem.at[b])
                nxt = i + NB - 1
                if nxt < CH:
                    nb_ = nxt % NB
                    if nxt - NB >= 0:
                        stores.pop(nxt - NB).wait()
                    gathers[nxt] = pltpu.async_copy(
                        x_hbm.at[idx_v.at[nxt]], rows_v.at[nb_], gsem.at[nb_])
            for i in sorted(stores):
                stores.pop(i).wait()
            return 0

        lax.fori_loop(0, NCH, chunk_body, 0)

    return kern(x2, idx2)


def _edge_body(g_ref, x_ref, ctr_ref, wb_ref, gam_ref, bet_ref, out_ref,
               *, R, O, OPAD):
    x_rows = x_ref[0]
    ctr = ctr_ref[0]
    gam = gam_ref[...][None, :]
    bet = bet_ref[...][None, :]
    inv = jnp.sqrt(jnp.float32(1.0 + EPS))

    e3 = g_ref[0]
    d3 = (e3 - x_rows[:, None, :]).astype(jnp.bfloat16)
    z3 = lax.dot_general(d3, wb_ref[...], (((2,), (0,)), ((), ())),
                         preferred_element_type=jnp.float32)
    t = (ctr[:, None, :] + z3) / inv * gam[None] + bet[None]
    t = jnp.where(t >= 0.0, t, 0.2 * t)
    acc = jnp.max(t, axis=1)

    if OPAD != O:
        acc = jnp.concatenate(
            [acc, jnp.zeros((R, OPAD - O), jnp.float32)], axis=1)
    out_ref[0] = acc


def _edge_tc(gath, x, ctr, wb, gam, bet, OPAD, *, R=512):
    B, N, _ = x.shape
    O = ctr.shape[2]
    vec = lambda n: pl.BlockSpec((n,), lambda b, r: (0,))
    return pl.pallas_call(
        functools.partial(_edge_body, R=R, O=O, OPAD=OPAD),
        grid=(B, N // R),
        in_specs=[
            pl.BlockSpec((1, R, K, CP), lambda b, r: (b, r, 0, 0)),
            pl.BlockSpec((1, R, CP), lambda b, r: (b, r, 0)),
            pl.BlockSpec((1, R, O), lambda b, r: (b, r, 0)),
            pl.BlockSpec((CP, O), lambda b, r: (0, 0)),
            vec(O), vec(O),
        ],
        out_specs=pl.BlockSpec((1, R, OPAD), lambda b, r: (b, r, 0)),
        out_shape=jax.ShapeDtypeStruct((B, N, OPAD), jnp.float32),
    )(gath, x, ctr, wb, gam, bet)


def _head_body(x1_ref, x2_ref, x3_ref, x4_ref, w5_ref, a5_ref, c5_ref,
               fc1_ref, fb1_ref, a6_ref, c6_ref,
               fc2_ref, fb2_ref, a7_ref, c7_ref, out_ref, *, N):
    xc = jnp.concatenate(
        [x1_ref[0][:, :64], x2_ref[0][:, :64],
         x3_ref[0][:, :128], x4_ref[0]], axis=1)
    h5 = lax.dot_general(xc.astype(jnp.bfloat16),
                         w5_ref[...].astype(jnp.bfloat16),
                         (((1,), (0,)), ((), ())),
                         preferred_element_type=jnp.float32)
    inv = jnp.sqrt(jnp.float32(1.0 + EPS))
    h5 = h5 / inv * a5_ref[...][None, :] + c5_ref[...][None, :]
    h5 = jnp.where(h5 >= 0.0, h5, 0.2 * h5)
    mx = jnp.max(h5, axis=0)
    mn = jnp.sum(h5, axis=0) * (1.0 / N)
    feat = jnp.concatenate([mx, mn])[None, :]
    h = lax.dot_general(feat.astype(jnp.bfloat16),
                        fc1_ref[...].astype(jnp.bfloat16),
                        (((1,), (0,)), ((), ())),
                        preferred_element_type=jnp.float32)
    h = (h + fb1_ref[...][None, :]) / inv * a6_ref[...][None, :] + c6_ref[...][None, :]
    h = jnp.where(h >= 0.0, h, 0.2 * h)
    o = lax.dot_general(h.astype(jnp.bfloat16),
                        fc2_ref[...].astype(jnp.bfloat16),
                        (((1,), (0,)), ((), ())),
                        preferred_element_type=jnp.float32)
    out_ref[0] = (o + fb2_ref[...][None, :]) / inv * a7_ref[...][None, :] + c7_ref[...][None, :]


def _head(x1, x2, x3, x4, w5t, a5, c5, fc1t, fc1_b, a6, c6, fc2t, fc2_b, a7, c7):
    B, N, _ = x1.shape
    vec = lambda n: pl.BlockSpec((n,), lambda b: (0,))
    return pl.pallas_call(
        functools.partial(_head_body, N=N),
        grid=(B,),
        in_specs=[
            pl.BlockSpec((1, N, x1.shape[2]), lambda b: (b, 0, 0)),
            pl.BlockSpec((1, N, x2.shape[2]), lambda b: (b, 0, 0)),
            pl.BlockSpec((1, N, x3.shape[2]), lambda b: (b, 0, 0)),
            pl.BlockSpec((1, N, x4.shape[2]), lambda b: (b, 0, 0)),
            pl.BlockSpec((512, 1024), lambda b: (0, 0)),
            vec(1024), vec(1024),
            pl.BlockSpec((2048, 512), lambda b: (0, 0)),
            vec(512), vec(512), vec(512),
            pl.BlockSpec((512, 256), lambda b: (0, 0)),
            vec(256), vec(256), vec(256),
        ],
        out_specs=pl.BlockSpec((1, 1, 256), lambda b: (b, 0, 0)),
        out_shape=jax.ShapeDtypeStruct((B, 1, 256), jnp.float32),
    )(x1, x2, x3, x4, w5t, a5, c5, fc1t, fc1_b, a6, c6, fc2t, fc2_b, a7, c7).reshape(B, 256)


def _edge_layer(x, W, g, b, OPAD):
    B, N, _ = x.shape
    O, C2 = W.shape
    C = C2 // 2
    wa = jnp.pad(jnp.transpose(W[:, :C]), ((0, CP - C), (0, 0))).astype(jnp.bfloat16)
    wb = jnp.pad(jnp.transpose(W[:, C:]), ((0, CP - C), (0, 0))).astype(jnp.bfloat16)
    idx, ctr = _knn_proj(x, wa)
    gath = _sc_gather(x.reshape(B * N, CP), idx.reshape(B * N, K))
    return _edge_tc(gath.reshape(B, N, K, CP), x, ctr, wb, g, b, OPAD)


def kernel(points, W1, g1, b1, W2, g2, b2, W3, g3, b3, W4, g4, b4,
           W5, g5, b5, fc1_W, fc1_b, g6, b6, fc2_W, fc2_b, g7, b7):
    B, N, _ = points.shape
    x0 = jnp.pad(points[:, :, :3].astype(jnp.float32),
                 ((0, 0), (0, 0), (0, CP - 3)))

    x1l, x2l, x3l, x4l = [], [], [], []
    for bi in range(B):
        xb = x0[bi:bi + 1]
        xb1 = _edge_layer(xb, W1, g1, b1, 128)
        xb2 = _edge_layer(xb1, W2, g2, b2, 128)
        xb3 = _edge_layer(xb2, W3, g3, b3, 128)
        xb4 = _edge_layer(xb3, W4, g4, b4, 256)
        x1l.append(xb1); x2l.append(xb2); x3l.append(xb3); x4l.append(xb4)
    x1 = jnp.concatenate(x1l, axis=0)
    x2 = jnp.concatenate(x2l, axis=0)
    x3 = jnp.concatenate(x3l, axis=0)
    x4 = jnp.concatenate(x4l, axis=0)

    return _head(
        x1, x2, x3, x4,
        jnp.transpose(W5), g5, b5,
        jnp.transpose(fc1_W), fc1_b, g6, b6,
        jnp.transpose(fc2_W), fc2_b, g7, b7,
    )
